# baseline (speedup 1.0000x reference)
import jax
import jax.numpy as jnp
from jax import lax
from jax.experimental import pallas as pl
from jax.experimental.pallas import tpu as pltpu
from jax.experimental.pallas import tpu_sc as plsc

N = 10000
E = 320000
D = 128
NC = 2
NS = 16
NW = NC * NS
CK = 64
NCH = 160
EPT = NCH * CK
DCK = 128
DNCH = EPT // DCK
EPAD = NW * EPT
RPT = 640
NPAD = NS * RPT

_sc_mesh = plsc.VectorSubcoreMesh(
    core_axis_name="c", subcore_axis_name="s", num_cores=NC, num_subcores=NS
)

_f32 = jnp.float32


def _deg_body(dst_hbm, out0, out1, zb, didx_all, ones, acc):
    c = lax.axis_index("c")
    s = lax.axis_index("s")
    cbase = (c * NS + s) * DNCH
    pltpu.sync_copy(dst_hbm.at[pl.ds(cbase, DNCH)], didx_all)
    zeros16 = jnp.zeros((16,), _f32)

    def zloop(i, carry):
        zb[pl.ds(i * 16, 16)] = zeros16
        return carry

    lax.fori_loop(0, RPT // 16, zloop, 0)
    ones16 = jnp.ones((16,), _f32)
    for i in range(DCK // 16):
        ones[pl.ds(i * 16, 16)] = ones16
    pltpu.sync_copy(zb, acc.at[pl.ds(s * RPT, RPT)])
    plsc.subcore_barrier()

    def chunk(j, carry):
        pltpu.sync_copy(ones, acc.at[didx_all.at[j]], add=True)
        return carry

    lax.fori_loop(0, DNCH, chunk, 0)
    plsc.subcore_barrier()

    @pl.when(jnp.logical_and(c == 0, s == 0))
    def _():
        pltpu.sync_copy(acc, out0)

    @pl.when(jnp.logical_and(c == 1, s == 0))
    def _():
        pltpu.sync_copy(acc, out1)


_deg_kernel = pl.kernel(
    _deg_body,
    out_type=(
        jax.ShapeDtypeStruct((NPAD,), _f32),
        jax.ShapeDtypeStruct((NPAD,), _f32),
    ),
    mesh=_sc_mesh,
    scratch_types=[
        pltpu.VMEM((RPT,), _f32),
        pltpu.VMEM((DNCH, DCK), jnp.int32),
        pltpu.VMEM((DCK,), _f32),
        pltpu.VMEM_SHARED((NPAD,), _f32),
    ],
)


def _agg_body(g_hbm, src_hbm, dst_hbm, out0, out1,
              sidx_all, didx0, didx1, didx2, didx3, rows0, rows1, rows2, rows3, acc,
              semg0, semg1, semg2, semg3, sems0, sems1, sems2, sems3,
              semd0, semd1, semd2, semd3):
    c = lax.axis_index("c")
    s = lax.axis_index("s")
    ebase = (c * NS + s) * EPT
    didx = (didx0, didx1, didx2, didx3)
    rows = (rows0, rows1, rows2, rows3)
    semg = (semg0, semg1, semg2, semg3)
    sems = (sems0, sems1, sems2, sems3)
    semd = (semd0, semd1, semd2, semd3)

    pltpu.sync_copy(src_hbm.at[pl.ds(ebase, EPT)], sidx_all)

    zeros16 = jnp.zeros((16,), _f32)

    def zloop(i, carry):
        r = i // 8
        q = i - r * 8
        rows0[r, pl.ds(q * 16, 16)] = zeros16
        rows1[r, pl.ds(q * 16, 16)] = zeros16
        return carry

    lax.fori_loop(0, CK * (D // 16), zloop, 0)
    for p in range(RPT // (2 * CK)):
        pltpu.sync_copy(rows0, acc.at[pl.ds(s * RPT + (2 * p) * CK, CK)])
        pltpu.sync_copy(rows1, acc.at[pl.ds(s * RPT + (2 * p + 1) * CK, CK)])

    def gidx(j):
        return sidx_all.at[pl.ds(j * CK, CK)]

    pltpu.async_copy(dst_hbm.at[pl.ds(ebase, CK)], didx0, semd0)
    pltpu.async_copy(dst_hbm.at[pl.ds(ebase + CK, CK)], didx1, semd1)
    pltpu.async_copy(g_hbm.at[gidx(0)], rows0, semg0)
    pltpu.async_copy(g_hbm.at[gidx(1)], rows1, semg1)
    plsc.subcore_barrier()

    def chunk(j, carry):
        b = lax.rem(j, 4)
        b2 = lax.rem(j + 2, 4)

        def slot(k):
            pltpu.make_async_copy(dst_hbm.at[pl.ds(ebase + j * CK, CK)], didx[k], semd[k]).wait()
            pltpu.make_async_copy(g_hbm.at[gidx(j)], rows[k], semg[k]).wait()
            pltpu.async_copy(rows[k], acc.at[didx[k]], sems[k], add=True)
            k2 = (k + 2) % 4

            @pl.when(j >= 2)
            def _():
                pltpu.make_async_copy(rows[k2], acc.at[didx[k2]], sems[k2]).wait()

            @pl.when(j + 2 < NCH)
            def _():
                pltpu.async_copy(dst_hbm.at[pl.ds(ebase + (j + 2) * CK, CK)], didx[k2], semd[k2])
                pltpu.async_copy(g_hbm.at[gidx(j + 2)], rows[k2], semg[k2])

        for k in range(4):
            @pl.when(b == k)
            def _():
                slot(k)

        return carry

    lax.fori_loop(0, NCH, chunk, 0)
    kA = (NCH - 2) % 4
    kB = (NCH - 1) % 4
    pltpu.make_async_copy(rows[kA], acc.at[didx[kA]], sems[kA]).wait()
    pltpu.make_async_copy(rows[kB], acc.at[didx[kB]], sems[kB]).wait()
    plsc.subcore_barrier()

    tail = N - 15 * RPT

    @pl.when(jnp.logical_and(c == 0, s < 15))
    def _():
        pltpu.sync_copy(acc.at[pl.ds(s * RPT, RPT)], out0.at[pl.ds(s * RPT, RPT)])

    @pl.when(jnp.logical_and(c == 0, s == 15))
    def _():
        pltpu.sync_copy(acc.at[pl.ds(15 * RPT, tail)], out0.at[pl.ds(15 * RPT, tail)])

    @pl.when(jnp.logical_and(c == 1, s < 15))
    def _():
        pltpu.sync_copy(acc.at[pl.ds(s * RPT, RPT)], out1.at[pl.ds(s * RPT, RPT)])

    @pl.when(jnp.logical_and(c == 1, s == 15))
    def _():
        pltpu.sync_copy(acc.at[pl.ds(15 * RPT, tail)], out1.at[pl.ds(15 * RPT, tail)])


_agg_kernel = pl.kernel(
    _agg_body,
    out_type=(
        jax.ShapeDtypeStruct((N, D), _f32),
        jax.ShapeDtypeStruct((N, D), _f32),
    ),
    mesh=_sc_mesh,
    scratch_types=(
        [pltpu.VMEM((EPT,), jnp.int32)]
        + [pltpu.VMEM((CK,), jnp.int32)] * 4
        + [pltpu.VMEM((CK, D), _f32)] * 4
        + [pltpu.VMEM_SHARED((NPAD, D), _f32)]
        + [pltpu.SemaphoreType.DMA] * 12
    ),
)


RB = 2000


def _k2_body(d0_ref, d1_ref, x_ref, w1_ref, g1_ref, dis_ref):
    deg = d0_ref[...] + d1_ref[...] + 1.0
    dis = lax.rsqrt(deg)
    h = jnp.dot(x_ref[...], w1_ref[...], preferred_element_type=_f32)
    g1_ref[...] = h * dis
    dis_ref[...] = dis


_k2 = pl.pallas_call(
    _k2_body,
    grid=(N // RB,),
    in_specs=[
        pl.BlockSpec((RB, 1), lambda i: (i, 0)),
        pl.BlockSpec((RB, 1), lambda i: (i, 0)),
        pl.BlockSpec((RB, D), lambda i: (i, 0)),
        pl.BlockSpec((D, D), lambda i: (0, 0)),
    ],
    out_specs=[
        pl.BlockSpec((RB, D), lambda i: (i, 0)),
        pl.BlockSpec((RB, 1), lambda i: (i, 0)),
    ],
    out_shape=[
        jax.ShapeDtypeStruct((N, D), _f32),
        jax.ShapeDtypeStruct((N, 1), _f32),
    ],
)


def _k4_body(a0_ref, a1_ref, g1_ref, dis_ref, b1_ref, u_ref, stats_ref, acc):
    i = pl.program_id(0)
    u = (a0_ref[...] + a1_ref[...] + g1_ref[...]) * dis_ref[...] + b1_ref[...]
    u_ref[...] = u

    @pl.when(i == 0)
    def _():
        acc[...] = jnp.zeros_like(acc)

    ssum = jnp.sum(u, axis=0, keepdims=True)
    ssq = jnp.sum(u * u, axis=0, keepdims=True)
    acc[...] += jnp.concatenate([ssum, ssq], axis=0)
    stats_ref[...] = acc[...]


_k4 = pl.pallas_call(
    _k4_body,
    grid=(N // RB,),
    in_specs=[
        pl.BlockSpec((RB, D), lambda i: (i, 0)),
        pl.BlockSpec((RB, D), lambda i: (i, 0)),
        pl.BlockSpec((RB, D), lambda i: (i, 0)),
        pl.BlockSpec((RB, 1), lambda i: (i, 0)),
        pl.BlockSpec((1, D), lambda i: (0, 0)),
    ],
    out_specs=[
        pl.BlockSpec((RB, D), lambda i: (i, 0)),
        pl.BlockSpec((2, D), lambda i: (0, 0)),
    ],
    out_shape=[
        jax.ShapeDtypeStruct((N, D), _f32),
        jax.ShapeDtypeStruct((2, D), _f32),
    ],
    scratch_shapes=[pltpu.VMEM((2, D), _f32)],
)


def _k5_body(u_ref, stats_ref, x_ref, dis_ref, gamma_ref, beta_ref, w2_ref, g2_ref):
    inv_n = 1.0 / N
    mean = stats_ref[0:1, :] * inv_n
    var = stats_ref[1:2, :] * inv_n - mean * mean
    inv = lax.rsqrt(var + 1e-5)
    bn = (u_ref[...] - mean) * (inv * gamma_ref[...]) + beta_ref[...]
    r = jnp.maximum(bn, 0.0) + x_ref[...]
    h2 = jnp.dot(r, w2_ref[...], preferred_element_type=_f32)
    g2_ref[...] = h2 * dis_ref[...]


_k5 = pl.pallas_call(
    _k5_body,
    grid=(N // RB,),
    in_specs=[
        pl.BlockSpec((RB, D), lambda i: (i, 0)),
        pl.BlockSpec((2, D), lambda i: (0, 0)),
        pl.BlockSpec((RB, D), lambda i: (i, 0)),
        pl.BlockSpec((RB, 1), lambda i: (i, 0)),
        pl.BlockSpec((1, D), lambda i: (0, 0)),
        pl.BlockSpec((1, D), lambda i: (0, 0)),
        pl.BlockSpec((D, D), lambda i: (0, 0)),
    ],
    out_specs=pl.BlockSpec((RB, D), lambda i: (i, 0)),
    out_shape=jax.ShapeDtypeStruct((N, D), _f32),
)


def _k7_body(a0_ref, a1_ref, g2_ref, dis_ref, b2_ref, out_ref):
    out_ref[...] = (a0_ref[...] + a1_ref[...] + g2_ref[...]) * dis_ref[...] + b2_ref[...]


_k7 = pl.pallas_call(
    _k7_body,
    grid=(N // RB,),
    in_specs=[
        pl.BlockSpec((RB, D), lambda i: (i, 0)),
        pl.BlockSpec((RB, D), lambda i: (i, 0)),
        pl.BlockSpec((RB, D), lambda i: (i, 0)),
        pl.BlockSpec((RB, 1), lambda i: (i, 0)),
        pl.BlockSpec((1, D), lambda i: (0, 0)),
    ],
    out_specs=pl.BlockSpec((RB, D), lambda i: (i, 0)),
    out_shape=jax.ShapeDtypeStruct((N, D), _f32),
)


def kernel(x, edge_index, W1, b1, gamma1, beta1, W2, b2):
    src = edge_index[0]
    dst = edge_index[1]
    pad = EPAD - E
    dummy_dst = N + jnp.arange(pad, dtype=jnp.int32) % (NPAD - N)
    dummy_src = jnp.arange(pad, dtype=jnp.int32) % N
    src_p1 = jnp.concatenate([src, dummy_src])
    dst_p1 = jnp.concatenate([dst, dummy_dst])
    dst_p2 = dst_p1.reshape(EPAD // DCK, DCK)
    d0, d1 = _deg_kernel(dst_p2)
    d0 = d0[:N].reshape(N, 1)
    d1 = d1[:N].reshape(N, 1)
    g1, dis = _k2(d0, d1, x, W1)
    a0, a1 = _agg_kernel(g1, src_p1, dst_p1)
    u, stats = _k4(a0, a1, g1, dis, b1.reshape(1, D))
    g2 = _k5(u, stats, x, dis, gamma1.reshape(1, D), beta1.reshape(1, D), W2)
    p0, p1 = _agg_kernel(g2, src_p1, dst_p1)
    out = _k7(p0, p1, g2, dis, b2.reshape(1, D))
    return out


# speedup vs baseline: 1.0867x; 1.0867x over previous
import jax
import jax.numpy as jnp
from jax import lax
from jax.experimental import pallas as pl
from jax.experimental.pallas import tpu as pltpu
from jax.experimental.pallas import tpu_sc as plsc

N = 10000
E = 320000
D = 128
NC = 2
NS = 16
NW = NC * NS
CK = 128
NCH = 80
EPT = NCH * CK
EPAD = NW * EPT
RPT = 640
NPAD = NS * RPT

_sc_mesh = plsc.VectorSubcoreMesh(
    core_axis_name="c", subcore_axis_name="s", num_cores=NC, num_subcores=NS
)

_f32 = jnp.float32


def _deg_body(dst_hbm, out0, out1, zb, didx_all, ones, acc):
    c = lax.axis_index("c")
    s = lax.axis_index("s")
    cbase = (c * NS + s) * NCH
    pltpu.sync_copy(dst_hbm.at[pl.ds(cbase, NCH)], didx_all)
    zeros16 = jnp.zeros((16,), _f32)

    def zloop(i, carry):
        zb[pl.ds(i * 16, 16)] = zeros16
        return carry

    lax.fori_loop(0, RPT // 16, zloop, 0)
    ones16 = jnp.ones((16,), _f32)
    for i in range(CK // 16):
        ones[pl.ds(i * 16, 16)] = ones16
    pltpu.sync_copy(zb, acc.at[pl.ds(s * RPT, RPT)])
    plsc.subcore_barrier()

    def chunk(j, carry):
        pltpu.sync_copy(ones, acc.at[didx_all.at[j]], add=True)
        return carry

    lax.fori_loop(0, NCH, chunk, 0)
    plsc.subcore_barrier()

    @pl.when(jnp.logical_and(c == 0, s == 0))
    def _():
        pltpu.sync_copy(acc, out0)

    @pl.when(jnp.logical_and(c == 1, s == 0))
    def _():
        pltpu.sync_copy(acc, out1)


_deg_kernel = pl.kernel(
    _deg_body,
    out_type=(
        jax.ShapeDtypeStruct((NPAD,), _f32),
        jax.ShapeDtypeStruct((NPAD,), _f32),
    ),
    mesh=_sc_mesh,
    scratch_types=[
        pltpu.VMEM((RPT,), _f32),
        pltpu.VMEM((NCH, CK), jnp.int32),
        pltpu.VMEM((CK,), _f32),
        pltpu.VMEM_SHARED((NPAD,), _f32),
    ],
)


def _agg_body(g_hbm, src_hbm, dst_hbm, out0, out1,
              sidx_all, didx0, didx1, rows0, rows1, acc, sem0, sem1, semd0, semd1):
    c = lax.axis_index("c")
    s = lax.axis_index("s")
    cbase = (c * NS + s) * NCH
    ebase = cbase * CK

    pltpu.sync_copy(src_hbm.at[pl.ds(cbase, NCH)], sidx_all)

    zeros16 = jnp.zeros((16,), _f32)

    def zloop(i, carry):
        r = i // 8
        q = i - r * 8
        rows0[r, pl.ds(q * 16, 16)] = zeros16
        return carry

    lax.fori_loop(0, CK * (D // 16), zloop, 0)
    for p in range(RPT // 128):
        pltpu.sync_copy(rows0, acc.at[pl.ds(s * RPT + p * 128, 128)])

    pltpu.async_copy(dst_hbm.at[pl.ds(ebase, CK)], didx0, semd0)
    pltpu.async_copy(dst_hbm.at[pl.ds(ebase + CK, CK)], didx1, semd1)
    pltpu.async_copy(g_hbm.at[sidx_all.at[0]], rows0, sem0)
    pltpu.async_copy(g_hbm.at[sidx_all.at[1]], rows1, sem1)
    plsc.subcore_barrier()

    def chunk(j, carry):
        b = lax.rem(j, 2)

        def do(didx, rows, sem, semd):
            pltpu.make_async_copy(dst_hbm.at[pl.ds(ebase + j * CK, CK)], didx, semd).wait()
            pltpu.make_async_copy(g_hbm.at[sidx_all.at[j]], rows, sem).wait()
            pltpu.sync_copy(rows, acc.at[didx], add=True)

            @pl.when(j + 2 < NCH)
            def _():
                pltpu.async_copy(dst_hbm.at[pl.ds(ebase + (j + 2) * CK, CK)], didx, semd)
                pltpu.async_copy(g_hbm.at[sidx_all.at[j + 2]], rows, sem)

        @pl.when(b == 0)
        def _():
            do(didx0, rows0, sem0, semd0)

        @pl.when(b == 1)
        def _():
            do(didx1, rows1, sem1, semd1)

        return carry

    lax.fori_loop(0, NCH, chunk, 0)
    plsc.subcore_barrier()

    tail = N - 15 * RPT

    @pl.when(jnp.logical_and(c == 0, s < 15))
    def _():
        pltpu.sync_copy(acc.at[pl.ds(s * RPT, RPT)], out0.at[pl.ds(s * RPT, RPT)])

    @pl.when(jnp.logical_and(c == 0, s == 15))
    def _():
        pltpu.sync_copy(acc.at[pl.ds(15 * RPT, tail)], out0.at[pl.ds(15 * RPT, tail)])

    @pl.when(jnp.logical_and(c == 1, s < 15))
    def _():
        pltpu.sync_copy(acc.at[pl.ds(s * RPT, RPT)], out1.at[pl.ds(s * RPT, RPT)])

    @pl.when(jnp.logical_and(c == 1, s == 15))
    def _():
        pltpu.sync_copy(acc.at[pl.ds(15 * RPT, tail)], out1.at[pl.ds(15 * RPT, tail)])


_agg_kernel = pl.kernel(
    _agg_body,
    out_type=(
        jax.ShapeDtypeStruct((N, D), _f32),
        jax.ShapeDtypeStruct((N, D), _f32),
    ),
    mesh=_sc_mesh,
    scratch_types=[
        pltpu.VMEM((NCH, CK), jnp.int32),
        pltpu.VMEM((CK,), jnp.int32),
        pltpu.VMEM((CK,), jnp.int32),
        pltpu.VMEM((CK, D), _f32),
        pltpu.VMEM((CK, D), _f32),
        pltpu.VMEM_SHARED((NPAD, D), _f32),
        pltpu.SemaphoreType.DMA,
        pltpu.SemaphoreType.DMA,
        pltpu.SemaphoreType.DMA,
        pltpu.SemaphoreType.DMA,
    ],
)


RB = 2000


def _k2_body(d0_ref, d1_ref, x_ref, w1_ref, g1_ref, dis_ref):
    deg = d0_ref[...] + d1_ref[...] + 1.0
    dis = lax.rsqrt(deg)
    h = jnp.dot(x_ref[...], w1_ref[...], preferred_element_type=_f32)
    g1_ref[...] = h * dis
    dis_ref[...] = dis


_k2 = pl.pallas_call(
    _k2_body,
    grid=(N // RB,),
    in_specs=[
        pl.BlockSpec((RB, 1), lambda i: (i, 0)),
        pl.BlockSpec((RB, 1), lambda i: (i, 0)),
        pl.BlockSpec((RB, D), lambda i: (i, 0)),
        pl.BlockSpec((D, D), lambda i: (0, 0)),
    ],
    out_specs=[
        pl.BlockSpec((RB, D), lambda i: (i, 0)),
        pl.BlockSpec((RB, 1), lambda i: (i, 0)),
    ],
    out_shape=[
        jax.ShapeDtypeStruct((N, D), _f32),
        jax.ShapeDtypeStruct((N, 1), _f32),
    ],
)


def _k4_body(a0_ref, a1_ref, g1_ref, dis_ref, b1_ref, u_ref, stats_ref, acc):
    i = pl.program_id(0)
    u = (a0_ref[...] + a1_ref[...] + g1_ref[...]) * dis_ref[...] + b1_ref[...]
    u_ref[...] = u

    @pl.when(i == 0)
    def _():
        acc[...] = jnp.zeros_like(acc)

    ssum = jnp.sum(u, axis=0, keepdims=True)
    ssq = jnp.sum(u * u, axis=0, keepdims=True)
    acc[...] += jnp.concatenate([ssum, ssq], axis=0)
    stats_ref[...] = acc[...]


_k4 = pl.pallas_call(
    _k4_body,
    grid=(N // RB,),
    in_specs=[
        pl.BlockSpec((RB, D), lambda i: (i, 0)),
        pl.BlockSpec((RB, D), lambda i: (i, 0)),
        pl.BlockSpec((RB, D), lambda i: (i, 0)),
        pl.BlockSpec((RB, 1), lambda i: (i, 0)),
        pl.BlockSpec((1, D), lambda i: (0, 0)),
    ],
    out_specs=[
        pl.BlockSpec((RB, D), lambda i: (i, 0)),
        pl.BlockSpec((2, D), lambda i: (0, 0)),
    ],
    out_shape=[
        jax.ShapeDtypeStruct((N, D), _f32),
        jax.ShapeDtypeStruct((2, D), _f32),
    ],
    scratch_shapes=[pltpu.VMEM((2, D), _f32)],
)


def _k5_body(u_ref, stats_ref, x_ref, dis_ref, gamma_ref, beta_ref, w2_ref, g2_ref):
    inv_n = 1.0 / N
    mean = stats_ref[0:1, :] * inv_n
    var = stats_ref[1:2, :] * inv_n - mean * mean
    inv = lax.rsqrt(var + 1e-5)
    bn = (u_ref[...] - mean) * (inv * gamma_ref[...]) + beta_ref[...]
    r = jnp.maximum(bn, 0.0) + x_ref[...]
    h2 = jnp.dot(r, w2_ref[...], preferred_element_type=_f32)
    g2_ref[...] = h2 * dis_ref[...]


_k5 = pl.pallas_call(
    _k5_body,
    grid=(N // RB,),
    in_specs=[
        pl.BlockSpec((RB, D), lambda i: (i, 0)),
        pl.BlockSpec((2, D), lambda i: (0, 0)),
        pl.BlockSpec((RB, D), lambda i: (i, 0)),
        pl.BlockSpec((RB, 1), lambda i: (i, 0)),
        pl.BlockSpec((1, D), lambda i: (0, 0)),
        pl.BlockSpec((1, D), lambda i: (0, 0)),
        pl.BlockSpec((D, D), lambda i: (0, 0)),
    ],
    out_specs=pl.BlockSpec((RB, D), lambda i: (i, 0)),
    out_shape=jax.ShapeDtypeStruct((N, D), _f32),
)


def _k7_body(a0_ref, a1_ref, g2_ref, dis_ref, b2_ref, out_ref):
    out_ref[...] = (a0_ref[...] + a1_ref[...] + g2_ref[...]) * dis_ref[...] + b2_ref[...]


_k7 = pl.pallas_call(
    _k7_body,
    grid=(N // RB,),
    in_specs=[
        pl.BlockSpec((RB, D), lambda i: (i, 0)),
        pl.BlockSpec((RB, D), lambda i: (i, 0)),
        pl.BlockSpec((RB, D), lambda i: (i, 0)),
        pl.BlockSpec((RB, 1), lambda i: (i, 0)),
        pl.BlockSpec((1, D), lambda i: (0, 0)),
    ],
    out_specs=pl.BlockSpec((RB, D), lambda i: (i, 0)),
    out_shape=jax.ShapeDtypeStruct((N, D), _f32),
)


def kernel(x, edge_index, W1, b1, gamma1, beta1, W2, b2):
    src = edge_index[0]
    dst = edge_index[1]
    pad = EPAD - E
    dummy_dst = N + jnp.arange(pad, dtype=jnp.int32) % (NPAD - N)
    dummy_src = jnp.arange(pad, dtype=jnp.int32) % N
    src_p2 = jnp.concatenate([src, dummy_src]).reshape(EPAD // CK, CK)
    dst_p1 = jnp.concatenate([dst, dummy_dst])
    dst_p2 = dst_p1.reshape(EPAD // CK, CK)
    d0, d1 = _deg_kernel(dst_p2)
    d0 = d0[:N].reshape(N, 1)
    d1 = d1[:N].reshape(N, 1)
    g1, dis = _k2(d0, d1, x, W1)
    a0, a1 = _agg_kernel(g1, src_p2, dst_p1)
    u, stats = _k4(a0, a1, g1, dis, b1.reshape(1, D))
    g2 = _k5(u, stats, x, dis, gamma1.reshape(1, D), beta1.reshape(1, D), W2)
    p0, p1 = _agg_kernel(g2, src_p2, dst_p1)
    out = _k7(p0, p1, g2, dis, b2.reshape(1, D))
    return out


# speedup vs baseline: 1.1043x; 1.0162x over previous
import jax
import jax.numpy as jnp
from jax import lax
from jax.experimental import pallas as pl
from jax.experimental.pallas import tpu as pltpu
from jax.experimental.pallas import tpu_sc as plsc

N = 10000
E = 320000
D = 128
NC = 2
NS = 16
NW = NC * NS
CK = 128
NCH = 80
EPT = NCH * CK
EPAD = NW * EPT
RPT = 640
NPAD = NS * RPT

_sc_mesh = plsc.VectorSubcoreMesh(
    core_axis_name="c", subcore_axis_name="s", num_cores=NC, num_subcores=NS
)

_f32 = jnp.float32


def _deg_body(dst_hbm, out0, out1, zb, didx_all, ones, acc, dsem):
    c = lax.axis_index("c")
    s = lax.axis_index("s")
    cbase = (c * NS + s) * NCH
    pltpu.sync_copy(dst_hbm.at[pl.ds(cbase, NCH)], didx_all)
    zeros16 = jnp.zeros((16,), _f32)

    def zloop(i, carry):
        zb[pl.ds(i * 16, 16)] = zeros16
        return carry

    lax.fori_loop(0, RPT // 16, zloop, 0)
    ones16 = jnp.ones((16,), _f32)
    for i in range(CK // 16):
        ones[pl.ds(i * 16, 16)] = ones16
    pltpu.sync_copy(zb, acc.at[pl.ds(s * RPT, RPT)])
    plsc.subcore_barrier()

    def chunk(j, carry):
        pltpu.async_copy(ones, acc.at[didx_all.at[j]], dsem, add=True)
        return carry

    lax.fori_loop(0, NCH, chunk, 0)

    def drain(j, carry):
        pltpu.make_async_copy(ones, acc.at[didx_all.at[j]], dsem).wait()
        return carry

    lax.fori_loop(0, NCH, drain, 0)
    plsc.subcore_barrier()

    @pl.when(jnp.logical_and(c == 0, s == 0))
    def _():
        pltpu.sync_copy(acc, out0)

    @pl.when(jnp.logical_and(c == 1, s == 0))
    def _():
        pltpu.sync_copy(acc, out1)


_deg_kernel = pl.kernel(
    _deg_body,
    out_type=(
        jax.ShapeDtypeStruct((NPAD,), _f32),
        jax.ShapeDtypeStruct((NPAD,), _f32),
    ),
    mesh=_sc_mesh,
    scratch_types=[
        pltpu.VMEM((RPT,), _f32),
        pltpu.VMEM((NCH, CK), jnp.int32),
        pltpu.VMEM((CK,), _f32),
        pltpu.VMEM_SHARED((NPAD,), _f32),
        pltpu.SemaphoreType.DMA,
    ],
)


def _agg_body(g_hbm, src_hbm, dst_hbm, out0, out1,
              sidx_all, didx0, didx1, rows0, rows1, acc, sem0, sem1, semd0, semd1):
    c = lax.axis_index("c")
    s = lax.axis_index("s")
    cbase = (c * NS + s) * NCH
    ebase = cbase * CK

    pltpu.sync_copy(src_hbm.at[pl.ds(cbase, NCH)], sidx_all)

    zeros16 = jnp.zeros((16,), _f32)

    def zloop(i, carry):
        r = i // 8
        q = i - r * 8
        rows0[r, pl.ds(q * 16, 16)] = zeros16
        return carry

    lax.fori_loop(0, CK * (D // 16), zloop, 0)
    for p in range(RPT // 128):
        pltpu.sync_copy(rows0, acc.at[pl.ds(s * RPT + p * 128, 128)])

    pltpu.async_copy(dst_hbm.at[pl.ds(ebase, CK)], didx0, semd0)
    pltpu.async_copy(dst_hbm.at[pl.ds(ebase + CK, CK)], didx1, semd1)
    pltpu.async_copy(g_hbm.at[sidx_all.at[0]], rows0, sem0)
    pltpu.async_copy(g_hbm.at[sidx_all.at[1]], rows1, sem1)
    plsc.subcore_barrier()

    def chunk(j, carry):
        b = lax.rem(j, 2)

        def do(didx, rows, sem, semd):
            pltpu.make_async_copy(dst_hbm.at[pl.ds(ebase + j * CK, CK)], didx, semd).wait()
            pltpu.make_async_copy(g_hbm.at[sidx_all.at[j]], rows, sem).wait()
            pltpu.sync_copy(rows, acc.at[didx], add=True)

            @pl.when(j + 2 < NCH)
            def _():
                pltpu.async_copy(dst_hbm.at[pl.ds(ebase + (j + 2) * CK, CK)], didx, semd)
                pltpu.async_copy(g_hbm.at[sidx_all.at[j + 2]], rows, sem)

        @pl.when(b == 0)
        def _():
            do(didx0, rows0, sem0, semd0)

        @pl.when(b == 1)
        def _():
            do(didx1, rows1, sem1, semd1)

        return carry

    lax.fori_loop(0, NCH, chunk, 0)
    plsc.subcore_barrier()

    tail = N - 15 * RPT

    @pl.when(jnp.logical_and(c == 0, s < 15))
    def _():
        pltpu.sync_copy(acc.at[pl.ds(s * RPT, RPT)], out0.at[pl.ds(s * RPT, RPT)])

    @pl.when(jnp.logical_and(c == 0, s == 15))
    def _():
        pltpu.sync_copy(acc.at[pl.ds(15 * RPT, tail)], out0.at[pl.ds(15 * RPT, tail)])

    @pl.when(jnp.logical_and(c == 1, s < 15))
    def _():
        pltpu.sync_copy(acc.at[pl.ds(s * RPT, RPT)], out1.at[pl.ds(s * RPT, RPT)])

    @pl.when(jnp.logical_and(c == 1, s == 15))
    def _():
        pltpu.sync_copy(acc.at[pl.ds(15 * RPT, tail)], out1.at[pl.ds(15 * RPT, tail)])


_agg_kernel = pl.kernel(
    _agg_body,
    out_type=(
        jax.ShapeDtypeStruct((N, D), _f32),
        jax.ShapeDtypeStruct((N, D), _f32),
    ),
    mesh=_sc_mesh,
    scratch_types=[
        pltpu.VMEM((NCH, CK), jnp.int32),
        pltpu.VMEM((CK,), jnp.int32),
        pltpu.VMEM((CK,), jnp.int32),
        pltpu.VMEM((CK, D), _f32),
        pltpu.VMEM((CK, D), _f32),
        pltpu.VMEM_SHARED((NPAD, D), _f32),
        pltpu.SemaphoreType.DMA,
        pltpu.SemaphoreType.DMA,
        pltpu.SemaphoreType.DMA,
        pltpu.SemaphoreType.DMA,
    ],
)


RB = 2000


def _k2_body(d0_ref, d1_ref, x_ref, w1_ref, g1_ref, dis_ref):
    deg = d0_ref[...] + d1_ref[...] + 1.0
    dis = lax.rsqrt(deg)
    h = jnp.dot(x_ref[...], w1_ref[...], preferred_element_type=_f32)
    g1_ref[...] = h * dis
    dis_ref[...] = dis


_k2 = pl.pallas_call(
    _k2_body,
    grid=(N // RB,),
    in_specs=[
        pl.BlockSpec((RB, 1), lambda i: (i, 0)),
        pl.BlockSpec((RB, 1), lambda i: (i, 0)),
        pl.BlockSpec((RB, D), lambda i: (i, 0)),
        pl.BlockSpec((D, D), lambda i: (0, 0)),
    ],
    out_specs=[
        pl.BlockSpec((RB, D), lambda i: (i, 0)),
        pl.BlockSpec((RB, 1), lambda i: (i, 0)),
    ],
    out_shape=[
        jax.ShapeDtypeStruct((N, D), _f32),
        jax.ShapeDtypeStruct((N, 1), _f32),
    ],
)


def _k4_body(a0_ref, a1_ref, g1_ref, dis_ref, b1_ref, u_ref, stats_ref, acc):
    i = pl.program_id(0)
    u = (a0_ref[...] + a1_ref[...] + g1_ref[...]) * dis_ref[...] + b1_ref[...]
    u_ref[...] = u

    @pl.when(i == 0)
    def _():
        acc[...] = jnp.zeros_like(acc)

    ssum = jnp.sum(u, axis=0, keepdims=True)
    ssq = jnp.sum(u * u, axis=0, keepdims=True)
    acc[...] += jnp.concatenate([ssum, ssq], axis=0)
    stats_ref[...] = acc[...]


_k4 = pl.pallas_call(
    _k4_body,
    grid=(N // RB,),
    in_specs=[
        pl.BlockSpec((RB, D), lambda i: (i, 0)),
        pl.BlockSpec((RB, D), lambda i: (i, 0)),
        pl.BlockSpec((RB, D), lambda i: (i, 0)),
        pl.BlockSpec((RB, 1), lambda i: (i, 0)),
        pl.BlockSpec((1, D), lambda i: (0, 0)),
    ],
    out_specs=[
        pl.BlockSpec((RB, D), lambda i: (i, 0)),
        pl.BlockSpec((2, D), lambda i: (0, 0)),
    ],
    out_shape=[
        jax.ShapeDtypeStruct((N, D), _f32),
        jax.ShapeDtypeStruct((2, D), _f32),
    ],
    scratch_shapes=[pltpu.VMEM((2, D), _f32)],
)


def _k5_body(u_ref, stats_ref, x_ref, dis_ref, gamma_ref, beta_ref, w2_ref, g2_ref):
    inv_n = 1.0 / N
    mean = stats_ref[0:1, :] * inv_n
    var = stats_ref[1:2, :] * inv_n - mean * mean
    inv = lax.rsqrt(var + 1e-5)
    bn = (u_ref[...] - mean) * (inv * gamma_ref[...]) + beta_ref[...]
    r = jnp.maximum(bn, 0.0) + x_ref[...]
    h2 = jnp.dot(r, w2_ref[...], preferred_element_type=_f32)
    g2_ref[...] = h2 * dis_ref[...]


_k5 = pl.pallas_call(
    _k5_body,
    grid=(N // RB,),
    in_specs=[
        pl.BlockSpec((RB, D), lambda i: (i, 0)),
        pl.BlockSpec((2, D), lambda i: (0, 0)),
        pl.BlockSpec((RB, D), lambda i: (i, 0)),
        pl.BlockSpec((RB, 1), lambda i: (i, 0)),
        pl.BlockSpec((1, D), lambda i: (0, 0)),
        pl.BlockSpec((1, D), lambda i: (0, 0)),
        pl.BlockSpec((D, D), lambda i: (0, 0)),
    ],
    out_specs=pl.BlockSpec((RB, D), lambda i: (i, 0)),
    out_shape=jax.ShapeDtypeStruct((N, D), _f32),
)


def _k7_body(a0_ref, a1_ref, g2_ref, dis_ref, b2_ref, out_ref):
    out_ref[...] = (a0_ref[...] + a1_ref[...] + g2_ref[...]) * dis_ref[...] + b2_ref[...]


_k7 = pl.pallas_call(
    _k7_body,
    grid=(N // RB,),
    in_specs=[
        pl.BlockSpec((RB, D), lambda i: (i, 0)),
        pl.BlockSpec((RB, D), lambda i: (i, 0)),
        pl.BlockSpec((RB, D), lambda i: (i, 0)),
        pl.BlockSpec((RB, 1), lambda i: (i, 0)),
        pl.BlockSpec((1, D), lambda i: (0, 0)),
    ],
    out_specs=pl.BlockSpec((RB, D), lambda i: (i, 0)),
    out_shape=jax.ShapeDtypeStruct((N, D), _f32),
)


def kernel(x, edge_index, W1, b1, gamma1, beta1, W2, b2):
    src = edge_index[0]
    dst = edge_index[1]
    pad = EPAD - E
    dummy_dst = N + jnp.arange(pad, dtype=jnp.int32) % (NPAD - N)
    dummy_src = jnp.arange(pad, dtype=jnp.int32) % N
    src_p2 = jnp.concatenate([src, dummy_src]).reshape(EPAD // CK, CK)
    dst_p1 = jnp.concatenate([dst, dummy_dst])
    dst_p2 = dst_p1.reshape(EPAD // CK, CK)
    d0, d1 = _deg_kernel(dst_p2)
    d0 = d0[:N].reshape(N, 1)
    d1 = d1[:N].reshape(N, 1)
    g1, dis = _k2(d0, d1, x, W1)
    a0, a1 = _agg_kernel(g1, src_p2, dst_p1)
    u, stats = _k4(a0, a1, g1, dis, b1.reshape(1, D))
    g2 = _k5(u, stats, x, dis, gamma1.reshape(1, D), beta1.reshape(1, D), W2)
    p0, p1 = _agg_kernel(g2, src_p2, dst_p1)
    out = _k7(p0, p1, g2, dis, b2.reshape(1, D))
    return out
